# baseline (speedup 1.0000x reference)
import functools

import jax
import jax.numpy as jnp
from jax import lax
from jax.experimental import pallas as pl
from jax.experimental.pallas import tpu as pltpu
from jax.experimental.pallas import tpu_sc as plsc

_NC = 2
_NS = 16
_NW = _NC * _NS
_LANES = 16


def _make_bulk_masked_copy(n, d):
    rows_per_w = (n // _NW) // 8 * 8
    rem = n - rows_per_w * _NW
    chunk = 80
    nsteps = rows_per_w // chunk
    assert rows_per_w % chunk == 0 and rem % 16 == 0 and rem <= 2 * chunk
    assert chunk % 16 == 0
    mesh = plsc.VectorSubcoreMesh(core_axis_name="c", subcore_axis_name="s")

    @functools.partial(
        pl.kernel,
        out_type=jax.ShapeDtypeStruct((n, d), jnp.float32),
        mesh=mesh,
        scratch_types=[
            pltpu.VMEM((2, chunk, d), jnp.float32),
            pltpu.VMEM((2, chunk), jnp.int32),
            pltpu.VMEM((d,), jnp.float32),
            pltpu.SemaphoreType.DMA,
            pltpu.SemaphoreType.DMA,
        ],
    )
    def bulk_masked_copy(x_hbm, flag_hbm, mask_hbm, out_hbm,
                         buf, fbuf, mask_v, rsem, wsem):
        wid = lax.axis_index("s") * _NC + lax.axis_index("c")
        base = wid * rows_per_w

        pltpu.sync_copy(mask_hbm, mask_v)

        def issue_read(i, b):
            pltpu.async_copy(
                x_hbm.at[pl.ds(base + i * chunk, chunk)], buf.at[b], rsem
            )
            pltpu.async_copy(
                flag_hbm.at[pl.ds(base + i * chunk, chunk)], fbuf.at[b], rsem
            )

        def wait_read(i, b):
            pltpu.make_async_copy(
                x_hbm.at[pl.ds(base + i * chunk, chunk)], buf.at[b], rsem
            ).wait()
            pltpu.make_async_copy(
                flag_hbm.at[pl.ds(base + i * chunk, chunk)], fbuf.at[b], rsem
            ).wait()

        def mask_rows(b, nrows):
            def grp_step(g, carry):
                fvec = fbuf[b, pl.ds(g * _LANES, _LANES)]
                for j in range(_LANES):
                    r = g * _LANES + j

                    @pl.when(fvec[j] != 0)
                    def _(r=r):
                        for kk in range(d // _LANES):
                            sl = pl.ds(kk * _LANES, _LANES)
                            buf[b, r, sl] = mask_v[sl]
                return carry

            lax.fori_loop(0, nrows // _LANES, grp_step, 0)

        issue_read(0, 0)

        def step(i, carry):
            p = i % 2
            wait_read(i, p)

            @pl.when(i >= 1)
            def _():
                pltpu.make_async_copy(
                    buf.at[1 - p],
                    out_hbm.at[pl.ds(base + (i - 1) * chunk, chunk)],
                    wsem,
                ).wait()

            @pl.when(i + 1 < nsteps)
            def _():
                issue_read(i + 1, 1 - p)

            mask_rows(p, chunk)
            pltpu.async_copy(
                buf.at[p], out_hbm.at[pl.ds(base + i * chunk, chunk)], wsem
            )
            return carry

        lax.fori_loop(0, nsteps, step, 0)
        last = nsteps - 1
        pltpu.make_async_copy(
            buf.at[last % 2], out_hbm.at[pl.ds(base + last * chunk, chunk)], wsem
        ).wait()

        @pl.when(wid == 0)
        def _():
            tail_base = rows_per_w * _NW
            h1 = min(chunk, rem)
            pltpu.sync_copy(x_hbm.at[pl.ds(tail_base, h1)], buf.at[0, pl.ds(0, h1)])
            pltpu.sync_copy(flag_hbm.at[pl.ds(tail_base, h1)], fbuf.at[0, pl.ds(0, h1)])
            mask_rows(0, h1)
            pltpu.sync_copy(buf.at[0, pl.ds(0, h1)], out_hbm.at[pl.ds(tail_base, h1)])
            if rem > chunk:
                h2 = rem - chunk
                pltpu.sync_copy(
                    x_hbm.at[pl.ds(tail_base + h1, h2)], buf.at[1, pl.ds(0, h2)]
                )
                pltpu.sync_copy(
                    flag_hbm.at[pl.ds(tail_base + h1, h2)], fbuf.at[1, pl.ds(0, h2)]
                )
                mask_rows(1, h2)
                pltpu.sync_copy(
                    buf.at[1, pl.ds(0, h2)], out_hbm.at[pl.ds(tail_base + h1, h2)]
                )

    return bulk_masked_copy


def _make_noise_patch(n, d, noise_chunk):
    mesh = plsc.VectorSubcoreMesh(core_axis_name="c", subcore_axis_name="s")

    @functools.partial(
        pl.kernel,
        out_type=(),
        mesh=mesh,
        scratch_types=[
            pltpu.VMEM((noise_chunk,), jnp.int32),
            pltpu.VMEM((noise_chunk,), jnp.int32),
            pltpu.VMEM((noise_chunk, d), jnp.float32),
            pltpu.SemaphoreType.DMA,
        ],
    )
    def noise_patch(x_hbm, nsrc_hbm, ndst_hbm, out_ref,
                    nsrc_v, ndst_v, rows_v, sem):
        wid = lax.axis_index("s") * _NC + lax.axis_index("c")
        base = wid * noise_chunk
        pltpu.sync_copy(nsrc_hbm.at[pl.ds(base, noise_chunk)], nsrc_v)
        pltpu.sync_copy(ndst_hbm.at[pl.ds(base, noise_chunk)], ndst_v)
        pltpu.async_copy(x_hbm.at[nsrc_v], rows_v, sem).wait()
        pltpu.async_copy(rows_v, out_ref.at[ndst_v], sem).wait()

    return noise_patch


def _pad_to(idx, total):
    k = idx.shape[0]
    return jnp.concatenate([idx, jnp.broadcast_to(idx[:1], (total - k,))])


def kernel(x, enc_mask_token, token_nodes, noise_nodes, noise_src, mask_nodes):
    n, d = x.shape
    k = noise_nodes.shape[0]

    flag = (
        jnp.zeros((n,), jnp.int32)
        .at[token_nodes]
        .set(1, unique_indices=True, mode="promise_in_bounds")
    )

    out = _make_bulk_masked_copy(n, d)(x, flag, enc_mask_token.reshape(d))

    noise_chunk = ((k + _NW - 1) // _NW + 7) // 8 * 8
    nsrc = _pad_to(noise_src, _NW * noise_chunk)
    ndst = _pad_to(noise_nodes, _NW * noise_chunk)

    out_ref = jax.new_ref(out)
    _make_noise_patch(n, d, noise_chunk)(x, nsrc, ndst, out_ref)
    return jax.freeze(out_ref)


# speedup vs baseline: 1.5593x; 1.5593x over previous
import functools

import jax
import jax.numpy as jnp
from jax import lax
from jax.experimental import pallas as pl
from jax.experimental.pallas import tpu as pltpu
from jax.experimental.pallas import tpu_sc as plsc

_NC = 2
_NS = 16
_NW = _NC * _NS


def _make_bulk_copy(n, d):
    rows_per_w = (n // _NW) // 8 * 8
    rem = n - rows_per_w * _NW
    chunk = 120
    nsteps = rows_per_w // chunk
    assert rows_per_w % chunk == 0 and rem % 8 == 0
    mesh = plsc.VectorSubcoreMesh(core_axis_name="c", subcore_axis_name="s")

    @functools.partial(
        pl.kernel,
        out_type=jax.ShapeDtypeStruct((n, d), jnp.float32),
        mesh=mesh,
        scratch_types=[
            pltpu.VMEM((2, chunk, d), jnp.float32),
            pltpu.SemaphoreType.DMA,
            pltpu.SemaphoreType.DMA,
        ],
    )
    def bulk_copy(x_hbm, out_hbm, buf, rsem, wsem):
        wid = lax.axis_index("s") * _NC + lax.axis_index("c")
        base = wid * rows_per_w

        def src_sl(i):
            return x_hbm.at[pl.ds(base + i * chunk, chunk)]

        def dst_sl(i):
            return out_hbm.at[pl.ds(base + i * chunk, chunk)]

        pltpu.async_copy(src_sl(0), buf.at[0], rsem)

        def step(i, carry):
            p = i % 2
            pltpu.make_async_copy(src_sl(i), buf.at[p], rsem).wait()

            @pl.when(i >= 1)
            def _():
                pltpu.make_async_copy(buf.at[1 - p], dst_sl(i - 1), wsem).wait()

            @pl.when(i + 1 < nsteps)
            def _():
                pltpu.async_copy(src_sl(i + 1), buf.at[1 - p], rsem)

            pltpu.async_copy(buf.at[p], dst_sl(i), wsem)
            return carry

        lax.fori_loop(0, nsteps, step, 0)
        last = nsteps - 1
        pltpu.make_async_copy(buf.at[last % 2], dst_sl(last), wsem).wait()

        @pl.when(wid == 0)
        def _():
            tail_base = rows_per_w * _NW
            h1 = min(chunk, rem)
            pltpu.sync_copy(x_hbm.at[pl.ds(tail_base, h1)], buf.at[0, pl.ds(0, h1)])
            pltpu.sync_copy(buf.at[0, pl.ds(0, h1)], out_hbm.at[pl.ds(tail_base, h1)])
            if rem > chunk:
                h2 = rem - chunk
                pltpu.sync_copy(
                    x_hbm.at[pl.ds(tail_base + h1, h2)], buf.at[1, pl.ds(0, h2)]
                )
                pltpu.sync_copy(
                    buf.at[1, pl.ds(0, h2)], out_hbm.at[pl.ds(tail_base + h1, h2)]
                )

    return bulk_copy


def _make_patch(n, d, tok_chunks, noise_chunk):
    mesh = plsc.VectorSubcoreMesh(core_axis_name="c", subcore_axis_name="s")

    @functools.partial(
        pl.kernel,
        out_type=(),
        mesh=mesh,
        scratch_types=[
            pltpu.VMEM((128, d), jnp.float32),
            pltpu.VMEM((tok_chunks, 128), jnp.int32),
            pltpu.VMEM((noise_chunk,), jnp.int32),
            pltpu.VMEM((noise_chunk,), jnp.int32),
            pltpu.VMEM((noise_chunk, d), jnp.float32),
            pltpu.SemaphoreType.DMA,
            pltpu.SemaphoreType.DMA,
        ],
    )
    def patch(x_hbm, mrep_hbm, tok_hbm, nsrc_hbm, ndst_hbm, out_ref,
              mrep_v, tidx_v, nsrc_v, ndst_v, rows_v, sem, tsem):
        wid = lax.axis_index("s") * _NC + lax.axis_index("c")

        pltpu.sync_copy(mrep_hbm, mrep_v)
        pltpu.sync_copy(tok_hbm.at[wid], tidx_v)

        for j in range(tok_chunks):
            pltpu.async_copy(mrep_v, out_ref.at[tidx_v.at[j]], tsem)

        base = wid * noise_chunk
        pltpu.sync_copy(nsrc_hbm.at[pl.ds(base, noise_chunk)], nsrc_v)
        pltpu.sync_copy(ndst_hbm.at[pl.ds(base, noise_chunk)], ndst_v)
        pltpu.async_copy(x_hbm.at[nsrc_v], rows_v, sem).wait()
        pltpu.async_copy(rows_v, out_ref.at[ndst_v], sem).wait()

        for j in range(tok_chunks):
            pltpu.make_async_copy(mrep_v, out_ref.at[tidx_v.at[j]], tsem).wait()

    return patch


def _pad_to(idx, total):
    k = idx.shape[0]
    return jnp.concatenate([idx, jnp.broadcast_to(idx[:1], (total - k,))])


def kernel(x, enc_mask_token, token_nodes, noise_nodes, noise_src, mask_nodes):
    n, d = x.shape
    t = token_nodes.shape[0]
    k = noise_nodes.shape[0]

    out = _make_bulk_copy(n, d)(x)

    tok_chunks = (t + _NW * 128 - 1) // (_NW * 128)
    tok = _pad_to(jnp.sort(token_nodes), _NW * 128 * tok_chunks)
    tok = tok.reshape(_NW, tok_chunks, 128)

    noise_chunk = ((k + _NW - 1) // _NW + 7) // 8 * 8
    nsrc = _pad_to(noise_src, _NW * noise_chunk)
    ndst = _pad_to(noise_nodes, _NW * noise_chunk)

    mrep = jnp.broadcast_to(enc_mask_token, (128, d))

    out_ref = jax.new_ref(out)
    _make_patch(n, d, tok_chunks, noise_chunk)(x, mrep, tok, nsrc, ndst, out_ref)
    return jax.freeze(out_ref)
